# Initial kernel scaffold; baseline (speedup 1.0000x reference)
import jax
import jax.numpy as jnp
from jax.experimental import pallas as pl


def kernel(x, y, li_w0, li_b0, li_w1, li_b1, li_w2, li_b2, li_w3, li_b3, li_w4, li_b4, li_w5, li_b5, lc_a_w0, lc_a_b0, lc_b_w0, lc_b_b0, lc_a_w1, lc_a_b1, lc_b_w1, lc_b_b1, lc_a_w2, lc_a_b2, lc_b_w2, lc_b_b2, lc_a_w3, lc_a_b3, lc_b_w3, lc_b_b3, lc_a_w4, lc_a_b4, lc_b_w4, lc_b_b4, lc_a_w5, lc_a_b5, lc_b_w5, lc_b_b5, ld_w0, ld_b0, ld_w1, ld_b1, ld_w2, ld_b2, ld_w3, ld_b3, ld_w4, ld_b4, dc_w, dc_b, df_w, df_b):
    raise NotImplementedError("write your pallas kernel here")


# speedup vs baseline: 12.1410x; 12.1410x over previous
import functools

import jax
import jax.numpy as jnp
from jax.experimental import pallas as pl
from jax.experimental.pallas import tpu as pltpu

_STRIDES = (1, 2, 2, 2, 2, 2)
_F32 = jnp.float32
_BF16 = jnp.bfloat16


def _pick_r(ho, wo):
    return min(ho, max(1, 1024 // wo))


def _cparams():
    return pltpu.CompilerParams(
        dimension_semantics=("parallel", "arbitrary"),
        vmem_limit_bytes=52 * 1024 * 1024,
    )


def _tapconv_body(*args, kh_n, kw_n, r_blk, act, n_in):
    x_refs = args[:n_in]
    w_refs = args[n_in:2 * n_in]
    b_ref = args[2 * n_in]
    o_ref = args[2 * n_in + 1]
    r = pl.program_id(1)
    wo = o_ref.shape[2]
    co = o_ref.shape[3]
    m = r_blk * wo
    acc = jnp.zeros((m, co), _F32) + b_ref[...]
    for x_ref, w_ref in zip(x_refs, w_refs):
        cin = x_ref.shape[3]
        slab = x_ref[0, pl.ds(r * r_blk, r_blk + kh_n - 1)]
        for kh in range(kh_n):
            for kw in range(kw_n):
                patch = slab[kh:kh + r_blk, kw:kw + wo, :].reshape(m, cin)
                acc = acc + jnp.dot(patch, w_ref[kh * kw_n + kw],
                                    preferred_element_type=_F32)
    if act == "relu":
        acc = jnp.maximum(acc, 0.0)
    elif act == "relu3_sigmoid1":
        lane = jax.lax.broadcasted_iota(jnp.int32, acc.shape, 1)
        acc = jnp.where(lane < 3, jnp.maximum(acc, 0.0), jax.nn.sigmoid(acc))
    o_ref[0] = acc.reshape(r_blk, wo, co).astype(o_ref.dtype)


def _tapconv(xs, ws, b, kh_n, kw_n, ho, wo, act, out_dtype=_BF16):
    n_in = len(xs)
    bsz = xs[0].shape[0]
    co = ws[0].shape[2]
    r_blk = _pick_r(ho, wo)
    grid = (bsz, ho // r_blk)
    in_specs = (
        [pl.BlockSpec((1,) + x.shape[1:], lambda bb, rr: (bb, 0, 0, 0)) for x in xs]
        + [pl.BlockSpec(w.shape, lambda bb, rr: (0, 0, 0)) for w in ws]
        + [pl.BlockSpec((1, co), lambda bb, rr: (0, 0))]
    )
    return pl.pallas_call(
        functools.partial(_tapconv_body, kh_n=kh_n, kw_n=kw_n, r_blk=r_blk,
                          act=act, n_in=n_in),
        out_shape=jax.ShapeDtypeStruct((bsz, ho, wo, co), out_dtype),
        grid=grid,
        in_specs=in_specs,
        out_specs=pl.BlockSpec((1, r_blk, wo, co), lambda bb, rr: (bb, rr, 0, 0)),
        compiler_params=_cparams(),
    )(*xs, *ws, b.reshape(1, co).astype(_F32))


def _poly_body(*args, r_blk, n_in):
    x_refs = args[:n_in]
    w_refs = args[n_in:2 * n_in]
    b_ref = args[2 * n_in]
    o_ref = args[2 * n_in + 1]
    r = pl.program_id(1)
    wo = o_ref.shape[2] // 2
    co = o_ref.shape[3]
    m = r_blk * wo
    slabs = [x_ref[0, pl.ds(r * r_blk, r_blk + 2)] for x_ref in x_refs]
    rows = []
    for pr in (0, 1):
        cols = []
        for pc in (0, 1):
            acc = jnp.zeros((m, co), _F32) + b_ref[...]
            for slab, x_ref, w_ref in zip(slabs, x_refs, w_refs):
                cin = x_ref.shape[3]
                for a in (0, 1):
                    for bb in (0, 1):
                        patch = slab[pr + a:pr + a + r_blk,
                                     pc + bb:pc + bb + wo, :].reshape(m, cin)
                        idx = ((pr * 2 + pc) * 2 + a) * 2 + bb
                        acc = acc + jnp.dot(patch, w_ref[idx],
                                            preferred_element_type=_F32)
            acc = jnp.maximum(acc, 0.0)
            cols.append(acc.reshape(r_blk, wo, co).astype(o_ref.dtype))
        rows.append(jnp.stack(cols, axis=2).reshape(r_blk, 2 * wo, co))
    o_ref[0] = jnp.stack(rows, axis=1).reshape(2 * r_blk, 2 * wo, co)


def _poly_weights(w):
    sel = {(0, 0): (0,), (0, 1): (1, 2), (1, 0): (0, 1), (1, 1): (2,)}
    blocks = []
    for pr in (0, 1):
        for pc in (0, 1):
            for a in (0, 1):
                for bb in (0, 1):
                    acc = 0.0
                    for kh in sel[(pr, a)]:
                        for kw in sel[(pc, bb)]:
                            acc = acc + w[kh, kw]
                    blocks.append(acc)
    return jnp.stack(blocks).astype(_BF16)


def _deconv_poly(xs, w, b):
    bsz, hi, wi, _ = xs[0].shape
    co = w.shape[3]
    splits = []
    off = 0
    for x in xs:
        splits.append(_poly_weights(w[:, :, off:off + x.shape[3], :]))
        off += x.shape[3]
    xps = [jnp.pad(x, ((0, 0), (1, 1), (1, 1), (0, 0))) for x in xs]
    r_blk = _pick_r(hi, wi)
    grid = (bsz, hi // r_blk)
    n_in = len(xs)
    in_specs = (
        [pl.BlockSpec((1,) + x.shape[1:], lambda bb, rr: (bb, 0, 0, 0)) for x in xps]
        + [pl.BlockSpec(wp.shape, lambda bb, rr: (0, 0, 0)) for wp in splits]
        + [pl.BlockSpec((1, co), lambda bb, rr: (0, 0))]
    )
    return pl.pallas_call(
        functools.partial(_poly_body, r_blk=r_blk, n_in=n_in),
        out_shape=jax.ShapeDtypeStruct((bsz, 2 * hi, 2 * wi, co), _BF16),
        grid=grid,
        in_specs=in_specs,
        out_specs=pl.BlockSpec((1, 2 * r_blk, 2 * wi, co),
                               lambda bb, rr: (bb, rr, 0, 0)),
        compiler_params=_cparams(),
    )(*xps, *splits, b.reshape(1, co).astype(_F32))


def _conf_body(x_ref, y_ref, w1x_ref, w1y_ref, w2_ref, b1_ref, b2_ref, o_ref,
               *, r_blk, ho):
    r = pl.program_id(1)
    wo = o_ref.shape[2]
    ch = o_ref.shape[3]
    xs = x_ref[0, pl.ds(r * r_blk, r_blk + 4)]
    ys = y_ref[0, pl.ds(r * r_blk, r_blk + 4)]
    mh = (r_blk + 2) * (wo + 2)
    acc = jnp.zeros((mh, ch), _F32) + b1_ref[...]
    for kh in range(3):
        for kw in range(3):
            px = xs[kh:kh + r_blk + 2, kw:kw + wo + 2, :].reshape(mh, ch)
            py = ys[kh:kh + r_blk + 2, kw:kw + wo + 2, :].reshape(mh, ch)
            acc = acc + jnp.dot(px, w1x_ref[kh * 3 + kw],
                                preferred_element_type=_F32)
            acc = acc + jnp.dot(py, w1y_ref[kh * 3 + kw],
                                preferred_element_type=_F32)
    h = jnp.maximum(acc, 0.0).reshape(r_blk + 2, wo + 2, ch)
    gi = jax.lax.broadcasted_iota(jnp.int32, (r_blk + 2, wo + 2, 1), 0) + r * r_blk - 1
    gj = jax.lax.broadcasted_iota(jnp.int32, (r_blk + 2, wo + 2, 1), 1) - 1
    valid = (gi >= 0) & (gi < ho) & (gj >= 0) & (gj < wo)
    h = jnp.where(valid, h, 0.0).astype(_BF16)
    mc = r_blk * wo
    acc2 = jnp.zeros((mc, 8), _F32) + b2_ref[...]
    for kh in range(3):
        for kw in range(3):
            acc2 = acc2 + jnp.dot(h[kh:kh + r_blk, kw:kw + wo, :].reshape(mc, ch),
                                  w2_ref[kh * 3 + kw], preferred_element_type=_F32)
    c = jax.nn.sigmoid(acc2[:, 0:1])
    xv = xs[2:2 + r_blk, 2:2 + wo, :].reshape(mc, ch).astype(_F32)
    yv = ys[2:2 + r_blk, 2:2 + wo, :].reshape(mc, ch).astype(_F32)
    o_ref[0] = (c * xv + (1.0 - c) * yv).astype(o_ref.dtype).reshape(r_blk, wo, ch)


def _conf_gate(xi, yi, w1, b1, w2, b2):
    bsz, ho, wo, ch = xi.shape
    xp = jnp.pad(xi, ((0, 0), (2, 2), (2, 2), (0, 0)))
    yp = jnp.pad(yi, ((0, 0), (2, 2), (2, 2), (0, 0)))
    w1x = w1[:, :, :ch, :].reshape(9, ch, ch).astype(_BF16)
    w1y = w1[:, :, ch:, :].reshape(9, ch, ch).astype(_BF16)
    w2p = jnp.pad(w2.reshape(9, ch, 1), ((0, 0), (0, 0), (0, 7))).astype(_BF16)
    b2p = jnp.pad(b2.reshape(1, 1), ((0, 0), (0, 7))).astype(_F32)
    r_blk = _pick_r(ho, wo)
    grid = (bsz, ho // r_blk)
    full = lambda arr: pl.BlockSpec((1,) + arr.shape[1:],
                                    lambda bb, rr: (bb, 0, 0, 0))
    return pl.pallas_call(
        functools.partial(_conf_body, r_blk=r_blk, ho=ho),
        out_shape=jax.ShapeDtypeStruct((bsz, ho, wo, ch), _BF16),
        grid=grid,
        in_specs=[
            full(xp), full(yp),
            pl.BlockSpec(w1x.shape, lambda bb, rr: (0, 0, 0)),
            pl.BlockSpec(w1y.shape, lambda bb, rr: (0, 0, 0)),
            pl.BlockSpec(w2p.shape, lambda bb, rr: (0, 0, 0)),
            pl.BlockSpec((1, ch), lambda bb, rr: (0, 0)),
            pl.BlockSpec((1, 8), lambda bb, rr: (0, 0)),
        ],
        out_specs=pl.BlockSpec((1, r_blk, wo, ch), lambda bb, rr: (bb, rr, 0, 0)),
        compiler_params=_cparams(),
    )(xp, yp, w1x, w1y, w2p, b1.reshape(1, ch).astype(_F32), b2p)


def _enc_first(x, w, b):
    xp = jnp.pad(x, ((0, 0), (1, 2), (1, 2), (0, 0)))
    wcat = jnp.concatenate([xp[:, :, kw:kw + x.shape[2], :] for kw in range(4)],
                           axis=-1)
    cin, co = w.shape[2], w.shape[3]
    wt = jnp.stack([w[kh].reshape(4 * cin, co) for kh in range(4)]).astype(_BF16)
    return _tapconv([wcat], [wt], b, 4, 1, x.shape[1], x.shape[2], "relu")


def _enc_down(x, w, b):
    bsz, h, wd, ch = x.shape
    ho, wo = h // 2, wd // 2
    xp = jnp.pad(x, ((0, 0), (1, 1), (1, 1), (0, 0)))
    xs = xp.reshape(bsz, ho + 1, 2, wo + 1, 2, ch)
    xs = xs.transpose(0, 1, 3, 2, 4, 5).reshape(bsz, ho + 1, wo + 1, 4 * ch)
    co = w.shape[3]
    wt = w.reshape(2, 2, 2, 2, ch, co).transpose(0, 2, 1, 3, 4, 5)
    wt = wt.reshape(4, 4 * ch, co).astype(_BF16)
    return _tapconv([xs], [wt], b, 2, 2, ho, wo, "relu")


def _head(x, w, b):
    sel = ((0,), (1, 2), (3,))
    blocks = []
    for i in range(3):
        for j in range(3):
            acc = 0.0
            for kh in sel[i]:
                for kw in sel[j]:
                    acc = acc + w[kh, kw]
            blocks.append(acc)
    wt = jnp.pad(jnp.stack(blocks), ((0, 0), (0, 0), (0, 4))).astype(_BF16)
    bp = jnp.pad(b, (0, 4))
    xp = jnp.pad(x, ((0, 0), (1, 1), (1, 1), (0, 0)))
    return _tapconv([xp], [wt], bp, 3, 3, x.shape[1], x.shape[2],
                    "relu3_sigmoid1", out_dtype=_F32)


@jax.jit
def _forward(params, x_nchw, y_nchw):
    bsz = x_nchw.shape[0]
    x = jnp.transpose(x_nchw, (0, 2, 3, 1)).astype(_BF16)
    y = jnp.transpose(y_nchw, (0, 2, 3, 1)).astype(_BF16)
    xy = jnp.concatenate([x, y], axis=0)

    feats = []
    for (w, b), stride in zip(params["layers_i"], _STRIDES):
        if stride == 1:
            xy = _enc_first(xy, w, b)
        else:
            xy = _enc_down(xy, w, b)
        feats.append(xy)

    merges = []
    for idx, f in enumerate(feats):
        (w1, b1), (w2, b2) = params["layers_c"][idx]
        merges.append(_conf_gate(f[:bsz], f[bsz:], w1, b1, w2, b2))

    up = None
    for i, (w, b) in enumerate(params["layers_d"]):
        skip = merges[-i - 1]
        xs = [merges[-1]] if up is None else [up, skip]
        up = _deconv_poly(xs, w, b)

    wc, bc = params["deconv_color"]
    wf, bf_ = params["deconv_confidence"]
    head = _head(up, jnp.concatenate([wc, wf], axis=-1),
                 jnp.concatenate([bc, bf_], axis=0))
    colors = jnp.transpose(head[..., 0:3], (0, 3, 1, 2))
    confidence = jnp.transpose(head[..., 3:4], (0, 3, 1, 2))
    return colors, confidence


def kernel(x, y,
           li_w0, li_b0, li_w1, li_b1, li_w2, li_b2,
           li_w3, li_b3, li_w4, li_b4, li_w5, li_b5,
           lc_a_w0, lc_a_b0, lc_b_w0, lc_b_b0,
           lc_a_w1, lc_a_b1, lc_b_w1, lc_b_b1,
           lc_a_w2, lc_a_b2, lc_b_w2, lc_b_b2,
           lc_a_w3, lc_a_b3, lc_b_w3, lc_b_b3,
           lc_a_w4, lc_a_b4, lc_b_w4, lc_b_b4,
           lc_a_w5, lc_a_b5, lc_b_w5, lc_b_b5,
           ld_w0, ld_b0, ld_w1, ld_b1, ld_w2, ld_b2,
           ld_w3, ld_b3, ld_w4, ld_b4,
           dc_w, dc_b, df_w, df_b):
    params = {
        "layers_i": [(li_w0, li_b0), (li_w1, li_b1), (li_w2, li_b2),
                     (li_w3, li_b3), (li_w4, li_b4), (li_w5, li_b5)],
        "layers_c": [((lc_a_w0, lc_a_b0), (lc_b_w0, lc_b_b0)),
                     ((lc_a_w1, lc_a_b1), (lc_b_w1, lc_b_b1)),
                     ((lc_a_w2, lc_a_b2), (lc_b_w2, lc_b_b2)),
                     ((lc_a_w3, lc_a_b3), (lc_b_w3, lc_b_b3)),
                     ((lc_a_w4, lc_a_b4), (lc_b_w4, lc_b_b4)),
                     ((lc_a_w5, lc_a_b5), (lc_b_w5, lc_b_b5))],
        "layers_d": [(ld_w0, ld_b0), (ld_w1, ld_b1), (ld_w2, ld_b2),
                     (ld_w3, ld_b3), (ld_w4, ld_b4)],
        "deconv_color": (dc_w, dc_b),
        "deconv_confidence": (df_w, df_b),
    }
    return _forward(params, x, y)


# speedup vs baseline: 12.6026x; 1.0380x over previous
import functools

import jax
import jax.numpy as jnp
from jax.experimental import pallas as pl
from jax.experimental.pallas import tpu as pltpu

_STRIDES = (1, 2, 2, 2, 2, 2)
_F32 = jnp.float32
_BF16 = jnp.bfloat16


def _pick_r(ho, wo):
    return min(ho, max(1, 2048 // wo))


def _cparams():
    return pltpu.CompilerParams(
        dimension_semantics=("parallel", "arbitrary"),
        vmem_limit_bytes=52 * 1024 * 1024,
    )


def _tapconv_body(*args, kh_n, kw_n, r_blk, act, n_in):
    x_refs = args[:n_in]
    w_refs = args[n_in:2 * n_in]
    b_ref = args[2 * n_in]
    o_ref = args[2 * n_in + 1]
    r = pl.program_id(1)
    wo = o_ref.shape[2]
    co = o_ref.shape[3]
    m = r_blk * wo
    acc = jnp.zeros((m, co), _F32) + b_ref[...]
    for x_ref, w_ref in zip(x_refs, w_refs):
        cin = x_ref.shape[3]
        slab = x_ref[0, pl.ds(r * r_blk, r_blk + kh_n - 1)]
        for kh in range(kh_n):
            for kw in range(kw_n):
                patch = slab[kh:kh + r_blk, kw:kw + wo, :].reshape(m, cin)
                acc = acc + jnp.dot(patch, w_ref[kh * kw_n + kw],
                                    preferred_element_type=_F32)
    if act == "relu":
        acc = jnp.maximum(acc, 0.0)
    elif act == "relu3_sigmoid1":
        lane = jax.lax.broadcasted_iota(jnp.int32, acc.shape, 1)
        acc = jnp.where(lane < 3, jnp.maximum(acc, 0.0), jax.nn.sigmoid(acc))
    o_ref[0] = acc.reshape(r_blk, wo, co).astype(o_ref.dtype)


def _tapconv(xs, ws, b, kh_n, kw_n, ho, wo, act, out_dtype=_BF16):
    n_in = len(xs)
    bsz = xs[0].shape[0]
    co = ws[0].shape[2]
    r_blk = _pick_r(ho, wo)
    grid = (bsz, ho // r_blk)
    in_specs = (
        [pl.BlockSpec((1,) + x.shape[1:], lambda bb, rr: (bb, 0, 0, 0)) for x in xs]
        + [pl.BlockSpec(w.shape, lambda bb, rr: (0, 0, 0)) for w in ws]
        + [pl.BlockSpec((1, co), lambda bb, rr: (0, 0))]
    )
    return pl.pallas_call(
        functools.partial(_tapconv_body, kh_n=kh_n, kw_n=kw_n, r_blk=r_blk,
                          act=act, n_in=n_in),
        out_shape=jax.ShapeDtypeStruct((bsz, ho, wo, co), out_dtype),
        grid=grid,
        in_specs=in_specs,
        out_specs=pl.BlockSpec((1, r_blk, wo, co), lambda bb, rr: (bb, rr, 0, 0)),
        compiler_params=_cparams(),
    )(*xs, *ws, b.reshape(1, co).astype(_F32))


def _poly_body(*args, r_blk, n_in):
    x_refs = args[:n_in]
    w_refs = args[n_in:2 * n_in]
    b_ref = args[2 * n_in]
    o_ref = args[2 * n_in + 1]
    r = pl.program_id(1)
    wo = o_ref.shape[2] // 2
    co = o_ref.shape[3]
    m = r_blk * wo
    slabs = [x_ref[0, pl.ds(r * r_blk, r_blk + 2)] for x_ref in x_refs]
    rows = []
    for pr in (0, 1):
        cols = []
        for pc in (0, 1):
            acc = jnp.zeros((m, co), _F32) + b_ref[...]
            for slab, x_ref, w_ref in zip(slabs, x_refs, w_refs):
                cin = x_ref.shape[3]
                for a in (0, 1):
                    for bb in (0, 1):
                        patch = slab[pr + a:pr + a + r_blk,
                                     pc + bb:pc + bb + wo, :].reshape(m, cin)
                        idx = ((pr * 2 + pc) * 2 + a) * 2 + bb
                        acc = acc + jnp.dot(patch, w_ref[idx],
                                            preferred_element_type=_F32)
            acc = jnp.maximum(acc, 0.0)
            cols.append(acc.reshape(r_blk, wo, co).astype(o_ref.dtype))
        rows.append(jnp.stack(cols, axis=2).reshape(r_blk, 2 * wo, co))
    o_ref[0] = jnp.stack(rows, axis=1).reshape(2 * r_blk, 2 * wo, co)


def _poly_weights(w):
    sel = {(0, 0): (0,), (0, 1): (1, 2), (1, 0): (0, 1), (1, 1): (2,)}
    blocks = []
    for pr in (0, 1):
        for pc in (0, 1):
            for a in (0, 1):
                for bb in (0, 1):
                    acc = 0.0
                    for kh in sel[(pr, a)]:
                        for kw in sel[(pc, bb)]:
                            acc = acc + w[kh, kw]
                    blocks.append(acc)
    return jnp.stack(blocks).astype(_BF16)


def _deconv_poly(xs, w, b):
    bsz, hi, wi, _ = xs[0].shape
    co = w.shape[3]
    splits = []
    off = 0
    for x in xs:
        splits.append(_poly_weights(w[:, :, off:off + x.shape[3], :]))
        off += x.shape[3]
    xps = [jnp.pad(x, ((0, 0), (1, 1), (1, 1), (0, 0))) for x in xs]
    r_blk = _pick_r(hi, wi)
    grid = (bsz, hi // r_blk)
    n_in = len(xs)
    in_specs = (
        [pl.BlockSpec((1,) + x.shape[1:], lambda bb, rr: (bb, 0, 0, 0)) for x in xps]
        + [pl.BlockSpec(wp.shape, lambda bb, rr: (0, 0, 0)) for wp in splits]
        + [pl.BlockSpec((1, co), lambda bb, rr: (0, 0))]
    )
    return pl.pallas_call(
        functools.partial(_poly_body, r_blk=r_blk, n_in=n_in),
        out_shape=jax.ShapeDtypeStruct((bsz, 2 * hi, 2 * wi, co), _BF16),
        grid=grid,
        in_specs=in_specs,
        out_specs=pl.BlockSpec((1, 2 * r_blk, 2 * wi, co),
                               lambda bb, rr: (bb, rr, 0, 0)),
        compiler_params=_cparams(),
    )(*xps, *splits, b.reshape(1, co).astype(_F32))


def _conf_body(x_ref, y_ref, w1x_ref, w1y_ref, w2_ref, b1_ref, b2_ref, o_ref,
               *, r_blk, ho):
    r = pl.program_id(1)
    wo = o_ref.shape[2]
    ch = o_ref.shape[3]
    xs = x_ref[0, pl.ds(r * r_blk, r_blk + 4)]
    ys = y_ref[0, pl.ds(r * r_blk, r_blk + 4)]
    mh = (r_blk + 2) * (wo + 2)
    acc = jnp.zeros((mh, ch), _F32) + b1_ref[...]
    for kh in range(3):
        for kw in range(3):
            px = xs[kh:kh + r_blk + 2, kw:kw + wo + 2, :].reshape(mh, ch)
            py = ys[kh:kh + r_blk + 2, kw:kw + wo + 2, :].reshape(mh, ch)
            acc = acc + jnp.dot(px, w1x_ref[kh * 3 + kw],
                                preferred_element_type=_F32)
            acc = acc + jnp.dot(py, w1y_ref[kh * 3 + kw],
                                preferred_element_type=_F32)
    h = jnp.maximum(acc, 0.0).reshape(r_blk + 2, wo + 2, ch)
    gi = jax.lax.broadcasted_iota(jnp.int32, (r_blk + 2, wo + 2, 1), 0) + r * r_blk - 1
    gj = jax.lax.broadcasted_iota(jnp.int32, (r_blk + 2, wo + 2, 1), 1) - 1
    valid = (gi >= 0) & (gi < ho) & (gj >= 0) & (gj < wo)
    h = jnp.where(valid, h, 0.0).astype(_BF16)
    mc = r_blk * wo
    acc2 = jnp.zeros((mc, 8), _F32) + b2_ref[...]
    for kh in range(3):
        for kw in range(3):
            acc2 = acc2 + jnp.dot(h[kh:kh + r_blk, kw:kw + wo, :].reshape(mc, ch),
                                  w2_ref[kh * 3 + kw], preferred_element_type=_F32)
    c = jax.nn.sigmoid(acc2[:, 0:1])
    xv = xs[2:2 + r_blk, 2:2 + wo, :].reshape(mc, ch).astype(_F32)
    yv = ys[2:2 + r_blk, 2:2 + wo, :].reshape(mc, ch).astype(_F32)
    o_ref[0] = (c * xv + (1.0 - c) * yv).astype(o_ref.dtype).reshape(r_blk, wo, ch)


def _conf_gate(xi, yi, w1, b1, w2, b2):
    bsz, ho, wo, ch = xi.shape
    xp = jnp.pad(xi, ((0, 0), (2, 2), (2, 2), (0, 0)))
    yp = jnp.pad(yi, ((0, 0), (2, 2), (2, 2), (0, 0)))
    w1x = w1[:, :, :ch, :].reshape(9, ch, ch).astype(_BF16)
    w1y = w1[:, :, ch:, :].reshape(9, ch, ch).astype(_BF16)
    w2p = jnp.pad(w2.reshape(9, ch, 1), ((0, 0), (0, 0), (0, 7))).astype(_BF16)
    b2p = jnp.pad(b2.reshape(1, 1), ((0, 0), (0, 7))).astype(_F32)
    r_blk = _pick_r(ho, wo)
    grid = (bsz, ho // r_blk)
    full = lambda arr: pl.BlockSpec((1,) + arr.shape[1:],
                                    lambda bb, rr: (bb, 0, 0, 0))
    return pl.pallas_call(
        functools.partial(_conf_body, r_blk=r_blk, ho=ho),
        out_shape=jax.ShapeDtypeStruct((bsz, ho, wo, ch), _BF16),
        grid=grid,
        in_specs=[
            full(xp), full(yp),
            pl.BlockSpec(w1x.shape, lambda bb, rr: (0, 0, 0)),
            pl.BlockSpec(w1y.shape, lambda bb, rr: (0, 0, 0)),
            pl.BlockSpec(w2p.shape, lambda bb, rr: (0, 0, 0)),
            pl.BlockSpec((1, ch), lambda bb, rr: (0, 0)),
            pl.BlockSpec((1, 8), lambda bb, rr: (0, 0)),
        ],
        out_specs=pl.BlockSpec((1, r_blk, wo, ch), lambda bb, rr: (bb, rr, 0, 0)),
        compiler_params=_cparams(),
    )(xp, yp, w1x, w1y, w2p, b1.reshape(1, ch).astype(_F32), b2p)


def _enc_first(x, w, b):
    xp = jnp.pad(x, ((0, 0), (1, 2), (1, 2), (0, 0)))
    wcat = jnp.concatenate([xp[:, :, kw:kw + x.shape[2], :] for kw in range(4)],
                           axis=-1)
    cin, co = w.shape[2], w.shape[3]
    wt = jnp.stack([w[kh].reshape(4 * cin, co) for kh in range(4)]).astype(_BF16)
    return _tapconv([wcat], [wt], b, 4, 1, x.shape[1], x.shape[2], "relu")


def _enc_down(x, w, b):
    bsz, h, wd, ch = x.shape
    ho, wo = h // 2, wd // 2
    xp = jnp.pad(x, ((0, 0), (1, 1), (1, 1), (0, 0)))
    xs = xp.reshape(bsz, ho + 1, 2, wo + 1, 2, ch)
    xs = xs.transpose(0, 1, 3, 2, 4, 5).reshape(bsz, ho + 1, wo + 1, 4 * ch)
    co = w.shape[3]
    wt = w.reshape(2, 2, 2, 2, ch, co).transpose(0, 2, 1, 3, 4, 5)
    wt = wt.reshape(4, 4 * ch, co).astype(_BF16)
    return _tapconv([xs], [wt], b, 2, 2, ho, wo, "relu")


def _head(x, w, b):
    sel = ((0,), (1, 2), (3,))
    blocks = []
    for i in range(3):
        for j in range(3):
            acc = 0.0
            for kh in sel[i]:
                for kw in sel[j]:
                    acc = acc + w[kh, kw]
            blocks.append(acc)
    wt = jnp.pad(jnp.stack(blocks), ((0, 0), (0, 0), (0, 4))).astype(_BF16)
    bp = jnp.pad(b, (0, 4))
    xp = jnp.pad(x, ((0, 0), (1, 1), (1, 1), (0, 0)))
    return _tapconv([xp], [wt], bp, 3, 3, x.shape[1], x.shape[2],
                    "relu3_sigmoid1", out_dtype=_F32)


@jax.jit
def _forward(params, x_nchw, y_nchw):
    bsz = x_nchw.shape[0]
    x = jnp.transpose(x_nchw, (0, 2, 3, 1)).astype(_BF16)
    y = jnp.transpose(y_nchw, (0, 2, 3, 1)).astype(_BF16)
    xy = jnp.concatenate([x, y], axis=0)

    feats = []
    for (w, b), stride in zip(params["layers_i"], _STRIDES):
        if stride == 1:
            xy = _enc_first(xy, w, b)
        else:
            xy = _enc_down(xy, w, b)
        feats.append(xy)

    merges = []
    for idx, f in enumerate(feats):
        (w1, b1), (w2, b2) = params["layers_c"][idx]
        merges.append(_conf_gate(f[:bsz], f[bsz:], w1, b1, w2, b2))

    up = None
    for i, (w, b) in enumerate(params["layers_d"]):
        skip = merges[-i - 1]
        xs = [merges[-1]] if up is None else [up, skip]
        up = _deconv_poly(xs, w, b)

    wc, bc = params["deconv_color"]
    wf, bf_ = params["deconv_confidence"]
    head = _head(up, jnp.concatenate([wc, wf], axis=-1),
                 jnp.concatenate([bc, bf_], axis=0))
    colors = jnp.transpose(head[..., 0:3], (0, 3, 1, 2))
    confidence = jnp.transpose(head[..., 3:4], (0, 3, 1, 2))
    return colors, confidence


def kernel(x, y,
           li_w0, li_b0, li_w1, li_b1, li_w2, li_b2,
           li_w3, li_b3, li_w4, li_b4, li_w5, li_b5,
           lc_a_w0, lc_a_b0, lc_b_w0, lc_b_b0,
           lc_a_w1, lc_a_b1, lc_b_w1, lc_b_b1,
           lc_a_w2, lc_a_b2, lc_b_w2, lc_b_b2,
           lc_a_w3, lc_a_b3, lc_b_w3, lc_b_b3,
           lc_a_w4, lc_a_b4, lc_b_w4, lc_b_b4,
           lc_a_w5, lc_a_b5, lc_b_w5, lc_b_b5,
           ld_w0, ld_b0, ld_w1, ld_b1, ld_w2, ld_b2,
           ld_w3, ld_b3, ld_w4, ld_b4,
           dc_w, dc_b, df_w, df_b):
    params = {
        "layers_i": [(li_w0, li_b0), (li_w1, li_b1), (li_w2, li_b2),
                     (li_w3, li_b3), (li_w4, li_b4), (li_w5, li_b5)],
        "layers_c": [((lc_a_w0, lc_a_b0), (lc_b_w0, lc_b_b0)),
                     ((lc_a_w1, lc_a_b1), (lc_b_w1, lc_b_b1)),
                     ((lc_a_w2, lc_a_b2), (lc_b_w2, lc_b_b2)),
                     ((lc_a_w3, lc_a_b3), (lc_b_w3, lc_b_b3)),
                     ((lc_a_w4, lc_a_b4), (lc_b_w4, lc_b_b4)),
                     ((lc_a_w5, lc_a_b5), (lc_b_w5, lc_b_b5))],
        "layers_d": [(ld_w0, ld_b0), (ld_w1, ld_b1), (ld_w2, ld_b2),
                     (ld_w3, ld_b3), (ld_w4, ld_b4)],
        "deconv_color": (dc_w, dc_b),
        "deconv_confidence": (df_w, df_b),
    }
    return _forward(params, x, y)


# speedup vs baseline: 13.1160x; 1.0407x over previous
import functools

import jax
import jax.numpy as jnp
from jax.experimental import pallas as pl
from jax.experimental.pallas import tpu as pltpu

_STRIDES = (1, 2, 2, 2, 2, 2)
_F32 = jnp.float32
_BF16 = jnp.bfloat16


def _pick_r(ho, wo):
    return min(ho, max(1, 2048 // wo))


def _cparams():
    return pltpu.CompilerParams(
        dimension_semantics=("parallel", "arbitrary"),
        vmem_limit_bytes=52 * 1024 * 1024,
    )


def _tapconv_body(*args, kh_n, kw_n, r_blk, act, n_in):
    x_refs = args[:n_in]
    w_refs = args[n_in:2 * n_in]
    b_ref = args[2 * n_in]
    o_ref = args[2 * n_in + 1]
    r = pl.program_id(1)
    wo = o_ref.shape[2]
    co = o_ref.shape[3]
    m = r_blk * wo
    acc = jnp.zeros((m, co), _F32) + b_ref[...]
    for x_ref, w_ref in zip(x_refs, w_refs):
        cin = x_ref.shape[3]
        slab = x_ref[0, pl.ds(r * r_blk, r_blk + kh_n - 1)]
        for kh in range(kh_n):
            for kw in range(kw_n):
                patch = slab[kh:kh + r_blk, kw:kw + wo, :].reshape(m, cin)
                acc = acc + jnp.dot(patch, w_ref[kh * kw_n + kw],
                                    preferred_element_type=_F32)
    if act == "relu":
        acc = jnp.maximum(acc, 0.0)
    elif act == "relu3_sigmoid1":
        lane = jax.lax.broadcasted_iota(jnp.int32, acc.shape, 1)
        acc = jnp.where(lane < 3, jnp.maximum(acc, 0.0), jax.nn.sigmoid(acc))
    o_ref[0] = acc.reshape(r_blk, wo, co).astype(o_ref.dtype)


def _tapconv(xs, ws, b, kh_n, kw_n, ho, wo, act, out_dtype=_BF16):
    n_in = len(xs)
    bsz = xs[0].shape[0]
    co = ws[0].shape[2]
    r_blk = _pick_r(ho, wo)
    grid = (bsz, ho // r_blk)
    in_specs = (
        [pl.BlockSpec((1,) + x.shape[1:], lambda bb, rr: (bb, 0, 0, 0)) for x in xs]
        + [pl.BlockSpec(w.shape, lambda bb, rr: (0, 0, 0)) for w in ws]
        + [pl.BlockSpec((1, co), lambda bb, rr: (0, 0))]
    )
    return pl.pallas_call(
        functools.partial(_tapconv_body, kh_n=kh_n, kw_n=kw_n, r_blk=r_blk,
                          act=act, n_in=n_in),
        out_shape=jax.ShapeDtypeStruct((bsz, ho, wo, co), out_dtype),
        grid=grid,
        in_specs=in_specs,
        out_specs=pl.BlockSpec((1, r_blk, wo, co), lambda bb, rr: (bb, rr, 0, 0)),
        compiler_params=_cparams(),
    )(*xs, *ws, b.reshape(1, co).astype(_F32))


def _poly_body(*args, r_blk, n_in):
    x_refs = args[:n_in]
    w_refs = args[n_in:2 * n_in]
    b_ref = args[2 * n_in]
    o_ref = args[2 * n_in + 1]
    r = pl.program_id(1)
    wo = o_ref.shape[2] // 2
    co = o_ref.shape[3]
    m = r_blk * wo
    slabs = [x_ref[0, pl.ds(r * r_blk, r_blk + 2)] for x_ref in x_refs]
    patches = []
    for slab, x_ref in zip(slabs, x_refs):
        cin = x_ref.shape[3]
        patches.append([[slab[ro:ro + r_blk, cc:cc + wo, :].reshape(m, cin)
                         for cc in range(3)] for ro in range(3)])
    rows = []
    for pr in (0, 1):
        cols = []
        for pc in (0, 1):
            acc = jnp.zeros((m, co), _F32) + b_ref[...]
            for pch, w_ref in zip(patches, w_refs):
                for a in (0, 1):
                    for bb in (0, 1):
                        idx = ((pr * 2 + pc) * 2 + a) * 2 + bb
                        acc = acc + jnp.dot(pch[pr + a][pc + bb], w_ref[idx],
                                            preferred_element_type=_F32)
            acc = jnp.maximum(acc, 0.0)
            cols.append(acc.reshape(r_blk, wo, co).astype(o_ref.dtype))
        rows.append(jnp.stack(cols, axis=2).reshape(r_blk, 2 * wo, co))
    o_ref[0] = jnp.stack(rows, axis=1).reshape(2 * r_blk, 2 * wo, co)


def _poly_weights(w):
    sel = {(0, 0): (0,), (0, 1): (1, 2), (1, 0): (0, 1), (1, 1): (2,)}
    blocks = []
    for pr in (0, 1):
        for pc in (0, 1):
            for a in (0, 1):
                for bb in (0, 1):
                    acc = 0.0
                    for kh in sel[(pr, a)]:
                        for kw in sel[(pc, bb)]:
                            acc = acc + w[kh, kw]
                    blocks.append(acc)
    return jnp.stack(blocks).astype(_BF16)


def _deconv_poly(xs, w, b):
    bsz, hi, wi, _ = xs[0].shape
    co = w.shape[3]
    splits = []
    off = 0
    for x in xs:
        splits.append(_poly_weights(w[:, :, off:off + x.shape[3], :]))
        off += x.shape[3]
    xps = [jnp.pad(x, ((0, 0), (1, 1), (1, 1), (0, 0))) for x in xs]
    r_blk = _pick_r(hi, wi)
    grid = (bsz, hi // r_blk)
    n_in = len(xs)
    in_specs = (
        [pl.BlockSpec((1,) + x.shape[1:], lambda bb, rr: (bb, 0, 0, 0)) for x in xps]
        + [pl.BlockSpec(wp.shape, lambda bb, rr: (0, 0, 0)) for wp in splits]
        + [pl.BlockSpec((1, co), lambda bb, rr: (0, 0))]
    )
    return pl.pallas_call(
        functools.partial(_poly_body, r_blk=r_blk, n_in=n_in),
        out_shape=jax.ShapeDtypeStruct((bsz, 2 * hi, 2 * wi, co), _BF16),
        grid=grid,
        in_specs=in_specs,
        out_specs=pl.BlockSpec((1, 2 * r_blk, 2 * wi, co),
                               lambda bb, rr: (bb, rr, 0, 0)),
        compiler_params=_cparams(),
    )(*xps, *splits, b.reshape(1, co).astype(_F32))


def _conf_body(xy_ref, w1_ref, w2_ref, b1_ref, b2_ref, o_ref, *, r_blk, ho):
    r = pl.program_id(1)
    wo = o_ref.shape[2]
    ch = o_ref.shape[3]
    xys = xy_ref[0, pl.ds(r * r_blk, r_blk + 4)]
    mh = (r_blk + 2) * (wo + 2)
    acc = jnp.zeros((mh, ch), _F32) + b1_ref[...]
    for kh in range(3):
        for kw in range(3):
            p = xys[kh:kh + r_blk + 2, kw:kw + wo + 2, :].reshape(mh, 2 * ch)
            acc = acc + jnp.dot(p, w1_ref[kh * 3 + kw],
                                preferred_element_type=_F32)
    h = jnp.maximum(acc, 0.0).reshape(r_blk + 2, wo + 2, ch)
    sh = (r_blk + 2, wo + 2, ch)
    gi = jax.lax.broadcasted_iota(jnp.int32, sh, 0) + r * r_blk - 1
    gj = jax.lax.broadcasted_iota(jnp.int32, sh, 1) - 1
    valid = (gi >= 0) & (gi < ho) & (gj >= 0) & (gj < wo)
    h = jnp.where(valid, h, 0.0).astype(_BF16)
    mc = r_blk * wo
    acc2 = jnp.zeros((mc, 8), _F32) + b2_ref[...]
    for kh in range(3):
        for kw in range(3):
            acc2 = acc2 + jnp.dot(h[kh:kh + r_blk, kw:kw + wo, :].reshape(mc, ch),
                                  w2_ref[kh * 3 + kw], preferred_element_type=_F32)
    c = jax.nn.sigmoid(acc2[:, 0:1])
    xv = xys[2:2 + r_blk, 2:2 + wo, :ch].reshape(mc, ch).astype(_F32)
    yv = xys[2:2 + r_blk, 2:2 + wo, ch:].reshape(mc, ch).astype(_F32)
    o_ref[0] = (c * xv + (1.0 - c) * yv).astype(o_ref.dtype).reshape(r_blk, wo, ch)


def _conf_gate(xi, yi, w1, b1, w2, b2):
    bsz, ho, wo, ch = xi.shape
    xyp = jnp.pad(jnp.concatenate([xi, yi], axis=-1),
                  ((0, 0), (2, 2), (2, 2), (0, 0)))
    w1f = w1.reshape(9, 2 * ch, ch).astype(_BF16)
    w2p = jnp.pad(w2.reshape(9, ch, 1), ((0, 0), (0, 0), (0, 7))).astype(_BF16)
    b2p = jnp.pad(b2.reshape(1, 1), ((0, 0), (0, 7))).astype(_F32)
    r_blk = _pick_r(ho, wo)
    grid = (bsz, ho // r_blk)
    return pl.pallas_call(
        functools.partial(_conf_body, r_blk=r_blk, ho=ho),
        out_shape=jax.ShapeDtypeStruct((bsz, ho, wo, ch), _BF16),
        grid=grid,
        in_specs=[
            pl.BlockSpec((1,) + xyp.shape[1:], lambda bb, rr: (bb, 0, 0, 0)),
            pl.BlockSpec(w1f.shape, lambda bb, rr: (0, 0, 0)),
            pl.BlockSpec(w2p.shape, lambda bb, rr: (0, 0, 0)),
            pl.BlockSpec((1, ch), lambda bb, rr: (0, 0)),
            pl.BlockSpec((1, 8), lambda bb, rr: (0, 0)),
        ],
        out_specs=pl.BlockSpec((1, r_blk, wo, ch), lambda bb, rr: (bb, rr, 0, 0)),
        compiler_params=_cparams(),
    )(xyp, w1f, w2p, b1.reshape(1, ch).astype(_F32), b2p)


def _enc_first(x, w, b):
    xp = jnp.pad(x, ((0, 0), (1, 2), (1, 2), (0, 0)))
    wcat = jnp.concatenate([xp[:, :, kw:kw + x.shape[2], :] for kw in range(4)],
                           axis=-1)
    cin, co = w.shape[2], w.shape[3]
    wt = jnp.stack([w[kh].reshape(4 * cin, co) for kh in range(4)]).astype(_BF16)
    return _tapconv([wcat], [wt], b, 4, 1, x.shape[1], x.shape[2], "relu")


def _enc_down(x, w, b):
    bsz, h, wd, ch = x.shape
    ho, wo = h // 2, wd // 2
    xp = jnp.pad(x, ((0, 0), (1, 1), (1, 1), (0, 0)))
    xs = xp.reshape(bsz, ho + 1, 2, wo + 1, 2, ch)
    xs = xs.transpose(0, 1, 3, 2, 4, 5).reshape(bsz, ho + 1, wo + 1, 4 * ch)
    co = w.shape[3]
    wt = w.reshape(2, 2, 2, 2, ch, co).transpose(0, 2, 1, 3, 4, 5)
    wt = wt.reshape(4, 4 * ch, co).astype(_BF16)
    return _tapconv([xs], [wt], b, 2, 2, ho, wo, "relu")


def _head(x, w, b):
    sel = ((0,), (1, 2), (3,))
    blocks = []
    for i in range(3):
        for j in range(3):
            acc = 0.0
            for kh in sel[i]:
                for kw in sel[j]:
                    acc = acc + w[kh, kw]
            blocks.append(acc)
    wt = jnp.pad(jnp.stack(blocks), ((0, 0), (0, 0), (0, 4))).astype(_BF16)
    bp = jnp.pad(b, (0, 4))
    xp = jnp.pad(x, ((0, 0), (1, 1), (1, 1), (0, 0)))
    return _tapconv([xp], [wt], bp, 3, 3, x.shape[1], x.shape[2],
                    "relu3_sigmoid1", out_dtype=_F32)


@jax.jit
def _forward(params, x_nchw, y_nchw):
    bsz = x_nchw.shape[0]
    x = jnp.transpose(x_nchw, (0, 2, 3, 1)).astype(_BF16)
    y = jnp.transpose(y_nchw, (0, 2, 3, 1)).astype(_BF16)
    xy = jnp.concatenate([x, y], axis=0)

    feats = []
    for (w, b), stride in zip(params["layers_i"], _STRIDES):
        if stride == 1:
            xy = _enc_first(xy, w, b)
        else:
            xy = _enc_down(xy, w, b)
        feats.append(xy)

    merges = []
    for idx, f in enumerate(feats):
        (w1, b1), (w2, b2) = params["layers_c"][idx]
        merges.append(_conf_gate(f[:bsz], f[bsz:], w1, b1, w2, b2))

    up = None
    for i, (w, b) in enumerate(params["layers_d"]):
        skip = merges[-i - 1]
        xs = [merges[-1]] if up is None else [up, skip]
        up = _deconv_poly(xs, w, b)

    wc, bc = params["deconv_color"]
    wf, bf_ = params["deconv_confidence"]
    head = _head(up, jnp.concatenate([wc, wf], axis=-1),
                 jnp.concatenate([bc, bf_], axis=0))
    colors = jnp.transpose(head[..., 0:3], (0, 3, 1, 2))
    confidence = jnp.transpose(head[..., 3:4], (0, 3, 1, 2))
    return colors, confidence


def kernel(x, y,
           li_w0, li_b0, li_w1, li_b1, li_w2, li_b2,
           li_w3, li_b3, li_w4, li_b4, li_w5, li_b5,
           lc_a_w0, lc_a_b0, lc_b_w0, lc_b_b0,
           lc_a_w1, lc_a_b1, lc_b_w1, lc_b_b1,
           lc_a_w2, lc_a_b2, lc_b_w2, lc_b_b2,
           lc_a_w3, lc_a_b3, lc_b_w3, lc_b_b3,
           lc_a_w4, lc_a_b4, lc_b_w4, lc_b_b4,
           lc_a_w5, lc_a_b5, lc_b_w5, lc_b_b5,
           ld_w0, ld_b0, ld_w1, ld_b1, ld_w2, ld_b2,
           ld_w3, ld_b3, ld_w4, ld_b4,
           dc_w, dc_b, df_w, df_b):
    params = {
        "layers_i": [(li_w0, li_b0), (li_w1, li_b1), (li_w2, li_b2),
                     (li_w3, li_b3), (li_w4, li_b4), (li_w5, li_b5)],
        "layers_c": [((lc_a_w0, lc_a_b0), (lc_b_w0, lc_b_b0)),
                     ((lc_a_w1, lc_a_b1), (lc_b_w1, lc_b_b1)),
                     ((lc_a_w2, lc_a_b2), (lc_b_w2, lc_b_b2)),
                     ((lc_a_w3, lc_a_b3), (lc_b_w3, lc_b_b3)),
                     ((lc_a_w4, lc_a_b4), (lc_b_w4, lc_b_b4)),
                     ((lc_a_w5, lc_a_b5), (lc_b_w5, lc_b_b5))],
        "layers_d": [(ld_w0, ld_b0), (ld_w1, ld_b1), (ld_w2, ld_b2),
                     (ld_w3, ld_b3), (ld_w4, ld_b4)],
        "deconv_color": (dc_w, dc_b),
        "deconv_confidence": (df_w, df_b),
    }
    return _forward(params, x, y)
